# baseline (speedup 1.0000x reference)
import functools

import jax
import jax.numpy as jnp
from jax import lax
from jax.experimental import pallas as pl
from jax.experimental.pallas import tpu as pltpu
from jax.experimental.pallas import tpu_sc as plsc

_B, _S, _D = 4, 8192, 1024
_L = 16


def _sc_kernel(emb_hbm, lm_hbm, out_hbm, mask_v, rows_v, sem):
    cid = lax.axis_index("c")
    sid = lax.axis_index("s")

    @pl.when(jnp.logical_and(cid == 0, sid < _B))
    def _():
        b = sid
        pltpu.sync_copy(lm_hbm.at[pl.ds(b * _S, _S)], mask_v)

        acc = mask_v[pl.ds(0, _L)]
        for i in range(1, _S // _L):
            acc = acc + mask_v[pl.ds(i * _L, _L)]
        total = acc[0]
        for i in range(1, _L):
            total = total + acc[i]

        idx_vec = jnp.full((_L,), total - 1, dtype=jnp.int32)
        pltpu.async_copy(emb_hbm.at[b].at[idx_vec], rows_v, sem).wait()
        pltpu.sync_copy(rows_v.at[0], out_hbm.at[b])


def kernel(embeddings, labels, embeddings_mask, labels_mask):
    del labels, embeddings_mask

    mesh = plsc.VectorSubcoreMesh(core_axis_name="c", subcore_axis_name="s", num_cores=1)
    run = functools.partial(
        pl.kernel,
        mesh=mesh,
        out_type=jax.ShapeDtypeStruct((_B, _D), jnp.float32),
        scratch_types=[
            pltpu.VMEM((_S,), jnp.int32),
            pltpu.VMEM((_L, _D), jnp.float32),
            pltpu.SemaphoreType.DMA,
        ],
    )(_sc_kernel)
    return run(embeddings, labels_mask.astype(jnp.int32).reshape(_B * _S))


# speedup vs baseline: 1.0118x; 1.0118x over previous
import functools

import jax
import jax.numpy as jnp
from jax import lax
from jax.experimental import pallas as pl
from jax.experimental.pallas import tpu as pltpu
from jax.experimental.pallas import tpu_sc as plsc

_B, _S, _D = 4, 8192, 1024
_L = 16


def _sc_kernel(emb_hbm, lm_hbm, out_hbm, mask_v, rows_v, sem):
    cid = lax.axis_index("c")
    sid = lax.axis_index("s")

    @pl.when(jnp.logical_and(cid == 0, sid < _B))
    def _():
        b = sid
        pltpu.sync_copy(lm_hbm.at[b], mask_v)

        acc = mask_v[pl.ds(0, _L)]
        for i in range(1, _S // _L):
            acc = acc + mask_v[pl.ds(i * _L, _L)]
        total = acc[0]
        for i in range(1, _L):
            total = total + acc[i]

        idx_vec = jnp.full((_L,), total - 1, dtype=jnp.int32)
        pltpu.async_copy(emb_hbm.at[b].at[idx_vec], rows_v, sem).wait()
        pltpu.sync_copy(rows_v.at[0], out_hbm.at[b])


def kernel(embeddings, labels, embeddings_mask, labels_mask):
    del labels, embeddings_mask

    mesh = plsc.VectorSubcoreMesh(core_axis_name="c", subcore_axis_name="s", num_cores=1)
    run = functools.partial(
        pl.kernel,
        mesh=mesh,
        out_type=jax.ShapeDtypeStruct((_B, _D), jnp.float32),
        scratch_types=[
            pltpu.VMEM((_S,), jnp.int32),
            pltpu.VMEM((_L, _D), jnp.float32),
            pltpu.SemaphoreType.DMA,
        ],
    )(_sc_kernel)
    return run(embeddings, labels_mask.astype(jnp.int32))
